# baseline (speedup 1.0000x reference)
import functools

import jax
import jax.numpy as jnp
from jax import lax
from jax.experimental import pallas as pl
from jax.experimental.pallas import tpu as pltpu
from jax.experimental.pallas import tpu_sc as plsc

B, S, H, FF, E, K = 1, 2048, 1024, 4096, 8, 2
T = B * S
TB = 256
NTB = T // TB
FFT = 512
NF = FF // FFT
EP = 128

BT = 512
NB = 16
NPAD = NB * BT
NW = 32
TT = T // NW
SLOT_T = NPAD // NW
L = 16

_f32 = jnp.float32
_i32 = jnp.int32


def _router_kernel(x_ref, wr1_ref, br1_ref, wr2t_ref, br2c_ref, lgt_ref):
    xb = x_ref[...]
    rh = jnp.maximum(
        jnp.dot(xb, wr1_ref[...], preferred_element_type=_f32)
        + br1_ref[...], 0.0)
    lgt = lax.dot_general(wr2t_ref[...], rh, (((1,), (1,)), ((), ())),
                          preferred_element_type=_f32)
    lgt_ref[...] = lgt + br2c_ref[...]


def _lane_iota():
    return lax.broadcasted_iota(_i32, (L,), 0)


def _lane_val(vec, e):
    return jnp.sum(jnp.where(_lane_iota() == e, vec, 0))


def _splat(x, dtype):
    return jnp.zeros((L,), dtype) + x


def _sc_route_kernel(lgt_hbm, x_hbm,
                     ws_hbm, inv_hbm, bexp_hbm, bval_hbm, xs_hbm,
                     lg_v, e1_v, e2_v, w1_v, w2_v, d1_v, d2_v,
                     st_v, wsl_v, bexp_v, bval_v, h1_v, h2_v, base_v,
                     gbufa_v, gbufb_v, sem, wsem):
    wid = lax.axis_index("s") * 2 + lax.axis_index("c")
    lane = _lane_iota()
    NCH = T // L

    def _zero(j, _):
        st_v[pl.ds(j * L, L)] = jnp.zeros((L,), _i32)
        wsl_v[pl.ds(j * L, L)] = jnp.zeros((L,), _f32)
        return _
    lax.fori_loop(0, NPAD // L, _zero, 0)

    pltpu.sync_copy(lgt_hbm, lg_v)

    neg = jnp.float32(-1e30)

    def _top2(c, cnt):
        ls = [lg_v[pl.ds(e * T + c * L, L)] for e in range(E)]
        m1 = ls[0]
        for e in range(1, E):
            m1 = jnp.maximum(m1, ls[e])
        i1 = jnp.full((L,), E, _i32)
        for e in reversed(range(E)):
            i1 = jnp.where(ls[e] == m1, e, i1)
        m2 = jnp.full((L,), neg, _f32)
        for e in range(E):
            m2 = jnp.maximum(m2, jnp.where(i1 == e, neg, ls[e]))
        i2 = jnp.full((L,), E, _i32)
        for e in reversed(range(E)):
            i2 = jnp.where((ls[e] == m2) & (i1 != e), e, i2)
        w1 = 1.0 / (1.0 + jnp.exp(m2 - m1))
        w2 = 1.0 - w1
        e1_v[pl.ds(c * L, L)] = i1
        e2_v[pl.ds(c * L, L)] = i2
        w1_v[pl.ds(c * L, L)] = w1
        w2_v[pl.ds(c * L, L)] = w2
        h1 = jnp.zeros((L,), _i32)
        h2 = jnp.zeros((L,), _i32)
        for e in range(E):
            n1 = jnp.sum(jnp.where(i1 == e, 1, 0))
            n2 = jnp.sum(jnp.where(i2 == e, 1, 0))
            h1 = h1 + jnp.where(lane == e, _splat(n1, _i32), 0)
            h2 = h2 + jnp.where(lane == e, _splat(n2, _i32), 0)
        h1_v[pl.ds(c * L, L)] = h1
        h2_v[pl.ds(c * L, L)] = h2
        return cnt + h1 + h2

    total = lax.fori_loop(0, NCH, _top2, jnp.zeros((L,), _i32))

    padded = (total + (BT - 1)) & ~(BT - 1)
    incl = plsc.cumsum(padded)
    excl = incl - padded
    tp = jnp.max(incl)

    tpv = _splat(tp, _i32)
    for j in range(NB // L):
        bstart = (lane + j * L) * BT
        eb = jnp.zeros((L,), _i32)
        for e in range(E):
            incl_e = _splat(_lane_val(incl, e), _i32)
            eb = eb + jnp.where(bstart >= incl_e, 1, 0)
        eb = jnp.minimum(eb, E - 1)
        bexp_v[pl.ds(j * L, L)] = eb
        bval_v[pl.ds(j * L, L)] = jnp.where(bstart < tpv, 1, 0)

    @pl.when(wid == 0)
    def _():
        pltpu.sync_copy(bexp_v, bexp_hbm)
        pltpu.sync_copy(bval_v, bval_hbm)

    def _dst(c, base):
        tok = c * L + lane
        for (ev_ref, wv_ref, dv_ref, hist_ref) in (
                (e1_v, w1_v, d1_v, h1_v), (e2_v, w2_v, d2_v, h2_v)):
            ev = ev_ref[pl.ds(c * L, L)]
            wv = wv_ref[pl.ds(c * L, L)]
            base_v[...] = base
            bsel = plsc.load_gather(base_v, [ev])
            pos = jnp.zeros((L,), _i32)
            for e in range(E):
                m = ev == e
                mi = jnp.where(m, 1, 0)
                pos = pos + jnp.where(m, plsc.cumsum(mi) - 1, 0)
            dst = bsel + pos
            dv_ref[pl.ds(c * L, L)] = dst
            plsc.store_scatter(st_v, [dst], tok)
            plsc.store_scatter(wsl_v, [dst], wv)
            base = base + hist_ref[pl.ds(c * L, L)]
        return base

    lax.fori_loop(0, NCH, _dst, excl)

    pltpu.sync_copy(d1_v.at[pl.ds(wid * TT, TT)],
                    inv_hbm.at[0, pl.ds(wid * TT, TT)])
    pltpu.sync_copy(d2_v.at[pl.ds(wid * TT, TT)],
                    inv_hbm.at[1, pl.ds(wid * TT, TT)])
    pltpu.sync_copy(wsl_v.at[pl.ds(wid * SLOT_T, SLOT_T)],
                    ws_hbm.at[pl.ds(wid * SLOT_T, SLOT_T)])

    GC = 32
    NG = SLOT_T // GC
    bufs = (gbufa_v, gbufb_v)

    def _start(g):
        return wid * SLOT_T + g * GC

    cp = pltpu.async_copy(x_hbm.at[st_v.at[pl.ds(_start(0), GC)]],
                          bufs[0], sem)
    wcps = []
    for g in range(NG):
        b = g & 1
        cp.wait()
        if g + 1 < NG:
            if g >= 1:
                wcps[g - 1].wait()
            cp = pltpu.async_copy(x_hbm.at[st_v.at[pl.ds(_start(g + 1), GC)]],
                                  bufs[1 - b], sem)
        wcps.append(pltpu.async_copy(bufs[b],
                                     xs_hbm.at[pl.ds(_start(g), GC)], wsem))
    wcps[NG - 2].wait()
    wcps[NG - 1].wait()


def _group_ffn_kernel(bexp_s, bval_s, xs_ref, w1_ref, b1_ref, w2_ref,
                      b2_ref, ws_ref, y_ref):
    b = pl.program_id(0)
    f = pl.program_id(1)

    @pl.when(bval_s[b] == 1)
    def _():
        h1 = (jnp.dot(xs_ref[...], w1_ref[0], preferred_element_type=_f32)
              + b1_ref[0])
        h1 = 0.5 * h1 * (1.0 + lax.erf(h1 * 0.7071067811865476))
        partial = jnp.dot(h1, w2_ref[0], preferred_element_type=_f32)

        @pl.when(f == 0)
        def _():
            y_ref[...] = partial

        @pl.when(f != 0)
        def _():
            y_ref[...] += partial

        @pl.when(f == NF - 1)
        def _():
            y_ref[...] = (y_ref[...] + b2_ref[0]) * ws_ref[...]


def _sc_combine_kernel(y_hbm, inv_hbm, out_hbm,
                       i0a_v, i0b_v, i1a_v, i1b_v, g0_v, g1_v, sem):
    wid = lax.axis_index("s") * 2 + lax.axis_index("c")
    t0 = wid * TT
    pltpu.sync_copy(inv_hbm.at[0, pl.ds(t0, 32)], i0a_v)
    pltpu.sync_copy(inv_hbm.at[0, pl.ds(t0 + 32, 32)], i0b_v)
    pltpu.sync_copy(inv_hbm.at[1, pl.ds(t0, 32)], i1a_v)
    pltpu.sync_copy(inv_hbm.at[1, pl.ds(t0 + 32, 32)], i1b_v)
    for g, (ia, ib) in enumerate(((i0a_v, i1a_v), (i0b_v, i1b_v))):
        pltpu.async_copy(y_hbm.at[ia], g0_v, sem).wait()
        pltpu.async_copy(y_hbm.at[ib], g1_v, sem).wait()
        for r in range(32):
            def _body(cc, _):
                for u in range(4):
                    sl = pl.ds(cc * 4 * L + u * L, L)
                    g0_v[r, sl] += g1_v[r, sl]
                return _
            lax.fori_loop(0, H // (4 * L), _body, 0)
        pltpu.sync_copy(g0_v, out_hbm.at[pl.ds(t0 + g * 32, 32)])


_SC_MESH = dict(core_axis_name="c", subcore_axis_name="s",
                num_cores=2, num_subcores=16)
_SC_PARAMS = pltpu.CompilerParams(needs_layout_passes=False)


def _sparse_moe(xf, Wr1, br1r, wr2t, br2c, W1, b1r, W2, b2r):
    lgt = pl.pallas_call(
        _router_kernel,
        grid=(NTB,),
        in_specs=[
            pl.BlockSpec((TB, H), lambda i: (i, 0)),
            pl.BlockSpec((H, H), lambda i: (0, 0)),
            pl.BlockSpec((1, H), lambda i: (0, 0)),
            pl.BlockSpec((E, H), lambda i: (0, 0)),
            pl.BlockSpec((E, 1), lambda i: (0, 0)),
        ],
        out_specs=pl.BlockSpec((E, TB), lambda i: (0, i)),
        out_shape=jax.ShapeDtypeStruct((E, T), _f32),
    )(xf, Wr1, br1r, wr2t, br2c)

    route = pl.kernel(
        _sc_route_kernel,
        out_type=(
            jax.ShapeDtypeStruct((NPAD,), _f32),
            jax.ShapeDtypeStruct((2, T), _i32),
            jax.ShapeDtypeStruct((NB,), _i32),
            jax.ShapeDtypeStruct((NB,), _i32),
            jax.ShapeDtypeStruct((NPAD, H), _f32),
        ),
        mesh=plsc.VectorSubcoreMesh(**_SC_MESH),
        compiler_params=_SC_PARAMS,
        scratch_types=[
            pltpu.VMEM((E * T,), _f32),
            pltpu.VMEM((T,), _i32),
            pltpu.VMEM((T,), _i32),
            pltpu.VMEM((T,), _f32),
            pltpu.VMEM((T,), _f32),
            pltpu.VMEM((T,), _i32),
            pltpu.VMEM((T,), _i32),
            pltpu.VMEM((NPAD,), _i32),
            pltpu.VMEM((NPAD,), _f32),
            pltpu.VMEM((NB,), _i32),
            pltpu.VMEM((NB,), _i32),
            pltpu.VMEM((T,), _i32),
            pltpu.VMEM((T,), _i32),
            pltpu.VMEM((L,), _i32),
            pltpu.VMEM((32, H), _f32),
            pltpu.VMEM((32, H), _f32),
            pltpu.SemaphoreType.DMA,
            pltpu.SemaphoreType.DMA,
        ],
    )
    ws, inv, bexp, bval, xs = route(lgt.reshape(E * T), xf)

    grid_spec = pltpu.PrefetchScalarGridSpec(
        num_scalar_prefetch=2,
        grid=(NB, NF),
        in_specs=[
            pl.BlockSpec((BT, H), lambda b, f, be, bv: (b * bv[b], 0)),
            pl.BlockSpec((1, H, FFT), lambda b, f, be, bv: (be[b], 0, f * bv[b])),
            pl.BlockSpec((1, 1, FFT),
                         lambda b, f, be, bv: (be[b] * NF + f * bv[b], 0, 0)),
            pl.BlockSpec((1, FFT, H), lambda b, f, be, bv: (be[b], f * bv[b], 0)),
            pl.BlockSpec((1, 1, H), lambda b, f, be, bv: (be[b], 0, 0)),
            pl.BlockSpec((BT, 1), lambda b, f, be, bv: (b * bv[b], 0)),
        ],
        out_specs=pl.BlockSpec((BT, H), lambda b, f, be, bv: (b, 0)),
    )
    y = pl.pallas_call(
        _group_ffn_kernel,
        grid_spec=grid_spec,
        out_shape=jax.ShapeDtypeStruct((NPAD, H), _f32),
    )(bexp, bval, xs, W1, b1r, W2, b2r, ws.reshape(NPAD, 1))

    combine = pl.kernel(
        _sc_combine_kernel,
        out_type=jax.ShapeDtypeStruct((T, H), _f32),
        mesh=plsc.VectorSubcoreMesh(**_SC_MESH),
        compiler_params=_SC_PARAMS,
        scratch_types=[
            pltpu.VMEM((32,), _i32),
            pltpu.VMEM((32,), _i32),
            pltpu.VMEM((32,), _i32),
            pltpu.VMEM((32,), _i32),
            pltpu.VMEM((32, H), _f32),
            pltpu.VMEM((32, H), _f32),
            pltpu.SemaphoreType.DMA,
        ],
    )
    out = combine(y, inv)
    return out


def kernel(x, Wr1, br1, Wr2, br2, W1, b1, W2, b2):
    xf = x.reshape(T, H)
    wr2t = Wr2.T
    br2c = br2.reshape(E, 1)
    br1r = br1.reshape(1, H)
    b1r = b1.reshape(E * NF, 1, FFT)
    b2r = b2.reshape(E, 1, H)
    out = _sparse_moe(xf, Wr1, br1r, wr2t, br2c, W1, b1r, W2, b2r)
    return out.reshape(B, S, H)


# speedup vs baseline: 1.1548x; 1.1548x over previous
import functools

import jax
import jax.numpy as jnp
from jax import lax
from jax.experimental import pallas as pl
from jax.experimental.pallas import tpu as pltpu
from jax.experimental.pallas import tpu_sc as plsc

B, S, H, FF, E, K = 1, 2048, 1024, 4096, 8, 2
T = B * S
TB = 256
NTB = T // TB
FFT = 512
NF = FF // FFT
EP = 128

BT = 512
NB = 16
NPAD = NB * BT
NW = 32
TT = T // NW
SLOT_T = NPAD // NW
L = 16

_f32 = jnp.float32
_i32 = jnp.int32


def _router_kernel(x_ref, wr1_ref, br1_ref, wr2t_ref, br2c_ref, lgt_ref):
    xb = x_ref[...]
    rh = jnp.maximum(
        jnp.dot(xb, wr1_ref[...], preferred_element_type=_f32)
        + br1_ref[...], 0.0)
    lgt = lax.dot_general(wr2t_ref[...], rh, (((1,), (1,)), ((), ())),
                          preferred_element_type=_f32)
    lgt_ref[...] = lgt + br2c_ref[...]


def _lane_iota():
    return lax.broadcasted_iota(_i32, (L,), 0)


def _lane_val(vec, e):
    return jnp.sum(jnp.where(_lane_iota() == e, vec, 0))


def _splat(x, dtype):
    return jnp.zeros((L,), dtype) + x


def _sc_route_kernel(lgt_hbm, x_hbm,
                     ws_hbm, inv_hbm, bexp_hbm, bval_hbm, xs_hbm,
                     lg_v, e1_v, e2_v, w1_v, w2_v, d1_v, d2_v,
                     st_v, wsl_v, bexp_v, bval_v, h1_v, h2_v, base_v,
                     gbufa_v, gbufb_v, sem, wsem):
    wid = lax.axis_index("s") * 2 + lax.axis_index("c")
    lane = _lane_iota()
    NCH = T // L

    def _zero(j, _):
        st_v[pl.ds(j * L, L)] = jnp.zeros((L,), _i32)
        wsl_v[pl.ds(j * L, L)] = jnp.zeros((L,), _f32)
        return _
    lax.fori_loop(0, NPAD // L, _zero, 0)

    pltpu.sync_copy(lgt_hbm, lg_v)

    neg = jnp.float32(-1e30)

    def _top2(c, cnt):
        ls = [lg_v[pl.ds(e * T + c * L, L)] for e in range(E)]
        m1 = ls[0]
        for e in range(1, E):
            m1 = jnp.maximum(m1, ls[e])
        i1 = jnp.full((L,), E, _i32)
        for e in reversed(range(E)):
            i1 = jnp.where(ls[e] == m1, e, i1)
        m2 = jnp.full((L,), neg, _f32)
        for e in range(E):
            m2 = jnp.maximum(m2, jnp.where(i1 == e, neg, ls[e]))
        i2 = jnp.full((L,), E, _i32)
        for e in reversed(range(E)):
            i2 = jnp.where((ls[e] == m2) & (i1 != e), e, i2)
        w1 = 1.0 / (1.0 + jnp.exp(m2 - m1))
        w2 = 1.0 - w1
        e1_v[pl.ds(c * L, L)] = i1
        e2_v[pl.ds(c * L, L)] = i2
        w1_v[pl.ds(c * L, L)] = w1
        w2_v[pl.ds(c * L, L)] = w2
        h1 = jnp.zeros((L,), _i32)
        h2 = jnp.zeros((L,), _i32)
        for e in range(E):
            n1 = jnp.sum(jnp.where(i1 == e, 1, 0))
            n2 = jnp.sum(jnp.where(i2 == e, 1, 0))
            h1 = h1 + jnp.where(lane == e, _splat(n1, _i32), 0)
            h2 = h2 + jnp.where(lane == e, _splat(n2, _i32), 0)
        h1_v[pl.ds(c * L, L)] = h1
        h2_v[pl.ds(c * L, L)] = h2
        return cnt + h1 + h2

    total = lax.fori_loop(0, NCH, _top2, jnp.zeros((L,), _i32))

    padded = (total + (BT - 1)) & ~(BT - 1)
    incl = plsc.cumsum(padded)
    excl = incl - padded
    tp = jnp.max(incl)

    tpv = _splat(tp, _i32)
    for j in range(NB // L):
        bstart = (lane + j * L) * BT
        eb = jnp.zeros((L,), _i32)
        for e in range(E):
            incl_e = _splat(_lane_val(incl, e), _i32)
            eb = eb + jnp.where(bstart >= incl_e, 1, 0)
        eb = jnp.minimum(eb, E - 1)
        bexp_v[pl.ds(j * L, L)] = eb
        bval_v[pl.ds(j * L, L)] = jnp.where(bstart < tpv, 1, 0)

    @pl.when(wid == 0)
    def _():
        pltpu.sync_copy(bexp_v, bexp_hbm)
        pltpu.sync_copy(bval_v, bval_hbm)

    def _dst(c, base):
        tok = c * L + lane
        for (ev_ref, wv_ref, dv_ref, hist_ref) in (
                (e1_v, w1_v, d1_v, h1_v), (e2_v, w2_v, d2_v, h2_v)):
            ev = ev_ref[pl.ds(c * L, L)]
            wv = wv_ref[pl.ds(c * L, L)]
            base_v[...] = base
            bsel = plsc.load_gather(base_v, [ev])
            pos = jnp.zeros((L,), _i32)
            for e in range(E):
                m = ev == e
                mi = jnp.where(m, 1, 0)
                pos = pos + jnp.where(m, plsc.cumsum(mi) - 1, 0)
            dst = bsel + pos
            dv_ref[pl.ds(c * L, L)] = dst
            plsc.store_scatter(st_v, [dst], tok)
            plsc.store_scatter(wsl_v, [dst], wv)
            base = base + hist_ref[pl.ds(c * L, L)]
        return base

    lax.fori_loop(0, NCH, _dst, excl)

    pltpu.sync_copy(d1_v.at[pl.ds(wid * TT, TT)],
                    inv_hbm.at[0, pl.ds(wid * TT, TT)])
    pltpu.sync_copy(d2_v.at[pl.ds(wid * TT, TT)],
                    inv_hbm.at[1, pl.ds(wid * TT, TT)])
    pltpu.sync_copy(wsl_v.at[pl.ds(wid * SLOT_T, SLOT_T)],
                    ws_hbm.at[pl.ds(wid * SLOT_T, SLOT_T)])

    for g in range(SLOT_T // 64):
        start = wid * SLOT_T + g * 64

        @pl.when(start < tp)
        def _():
            cpa = pltpu.async_copy(x_hbm.at[st_v.at[pl.ds(start, 32)]],
                                   gbufa_v, sem)
            cpb = pltpu.async_copy(x_hbm.at[st_v.at[pl.ds(start + 32, 32)]],
                                   gbufb_v, sem)
            cpa.wait()
            wa = pltpu.async_copy(gbufa_v, xs_hbm.at[pl.ds(start, 32)], wsem)
            cpb.wait()
            wb = pltpu.async_copy(gbufb_v, xs_hbm.at[pl.ds(start + 32, 32)],
                                  wsem)
            wa.wait()
            wb.wait()


def _group_ffn_kernel(bexp_s, bval_s, xs_ref, w1_ref, b1_ref, w2_ref,
                      b2_ref, ws_ref, y_ref):
    b = pl.program_id(0)
    f = pl.program_id(1)

    @pl.when(bval_s[b] == 1)
    def _():
        h1 = (jnp.dot(xs_ref[...], w1_ref[0], preferred_element_type=_f32)
              + b1_ref[0])
        h1 = 0.5 * h1 * (1.0 + lax.erf(h1 * 0.7071067811865476))
        partial = jnp.dot(h1, w2_ref[0], preferred_element_type=_f32)

        @pl.when(f == 0)
        def _():
            y_ref[...] = partial

        @pl.when(f != 0)
        def _():
            y_ref[...] += partial

        @pl.when(f == NF - 1)
        def _():
            y_ref[...] = (y_ref[...] + b2_ref[0]) * ws_ref[...]


def _sc_combine_kernel(y_hbm, inv_hbm, out_hbm,
                       i0a_v, i0b_v, i1a_v, i1b_v, g0_v, g1_v, sem):
    wid = lax.axis_index("s") * 2 + lax.axis_index("c")
    t0 = wid * TT
    pltpu.sync_copy(inv_hbm.at[0, pl.ds(t0, 32)], i0a_v)
    pltpu.sync_copy(inv_hbm.at[0, pl.ds(t0 + 32, 32)], i0b_v)
    pltpu.sync_copy(inv_hbm.at[1, pl.ds(t0, 32)], i1a_v)
    pltpu.sync_copy(inv_hbm.at[1, pl.ds(t0 + 32, 32)], i1b_v)
    for g, (ia, ib) in enumerate(((i0a_v, i1a_v), (i0b_v, i1b_v))):
        pltpu.async_copy(y_hbm.at[ia], g0_v, sem).wait()
        pltpu.async_copy(y_hbm.at[ib], g1_v, sem).wait()
        for r in range(32):
            def _body(cc, _):
                for u in range(4):
                    sl = pl.ds(cc * 4 * L + u * L, L)
                    g0_v[r, sl] += g1_v[r, sl]
                return _
            lax.fori_loop(0, H // (4 * L), _body, 0)
        pltpu.sync_copy(g0_v, out_hbm.at[pl.ds(t0 + g * 32, 32)])


_SC_MESH = dict(core_axis_name="c", subcore_axis_name="s",
                num_cores=2, num_subcores=16)
_SC_PARAMS = pltpu.CompilerParams(needs_layout_passes=False)


def _sparse_moe(xf, Wr1, br1r, wr2t, br2c, W1, b1r, W2, b2r):
    lgt = pl.pallas_call(
        _router_kernel,
        grid=(NTB,),
        in_specs=[
            pl.BlockSpec((TB, H), lambda i: (i, 0)),
            pl.BlockSpec((H, H), lambda i: (0, 0)),
            pl.BlockSpec((1, H), lambda i: (0, 0)),
            pl.BlockSpec((E, H), lambda i: (0, 0)),
            pl.BlockSpec((E, 1), lambda i: (0, 0)),
        ],
        out_specs=pl.BlockSpec((E, TB), lambda i: (0, i)),
        out_shape=jax.ShapeDtypeStruct((E, T), _f32),
    )(xf, Wr1, br1r, wr2t, br2c)

    route = pl.kernel(
        _sc_route_kernel,
        out_type=(
            jax.ShapeDtypeStruct((NPAD,), _f32),
            jax.ShapeDtypeStruct((2, T), _i32),
            jax.ShapeDtypeStruct((NB,), _i32),
            jax.ShapeDtypeStruct((NB,), _i32),
            jax.ShapeDtypeStruct((NPAD, H), _f32),
        ),
        mesh=plsc.VectorSubcoreMesh(**_SC_MESH),
        compiler_params=_SC_PARAMS,
        scratch_types=[
            pltpu.VMEM((E * T,), _f32),
            pltpu.VMEM((T,), _i32),
            pltpu.VMEM((T,), _i32),
            pltpu.VMEM((T,), _f32),
            pltpu.VMEM((T,), _f32),
            pltpu.VMEM((T,), _i32),
            pltpu.VMEM((T,), _i32),
            pltpu.VMEM((NPAD,), _i32),
            pltpu.VMEM((NPAD,), _f32),
            pltpu.VMEM((NB,), _i32),
            pltpu.VMEM((NB,), _i32),
            pltpu.VMEM((T,), _i32),
            pltpu.VMEM((T,), _i32),
            pltpu.VMEM((L,), _i32),
            pltpu.VMEM((32, H), _f32),
            pltpu.VMEM((32, H), _f32),
            pltpu.SemaphoreType.DMA,
            pltpu.SemaphoreType.DMA,
        ],
    )
    ws, inv, bexp, bval, xs = route(lgt.reshape(E * T), xf)

    grid_spec = pltpu.PrefetchScalarGridSpec(
        num_scalar_prefetch=2,
        grid=(NB, NF),
        in_specs=[
            pl.BlockSpec((BT, H), lambda b, f, be, bv: (b * bv[b], 0)),
            pl.BlockSpec((1, H, FFT), lambda b, f, be, bv: (be[b], 0, f * bv[b])),
            pl.BlockSpec((1, 1, FFT),
                         lambda b, f, be, bv: (be[b] * NF + f * bv[b], 0, 0)),
            pl.BlockSpec((1, FFT, H), lambda b, f, be, bv: (be[b], f * bv[b], 0)),
            pl.BlockSpec((1, 1, H), lambda b, f, be, bv: (be[b], 0, 0)),
            pl.BlockSpec((BT, 1), lambda b, f, be, bv: (b * bv[b], 0)),
        ],
        out_specs=pl.BlockSpec((BT, H), lambda b, f, be, bv: (b, 0)),
    )
    y = pl.pallas_call(
        _group_ffn_kernel,
        grid_spec=grid_spec,
        out_shape=jax.ShapeDtypeStruct((NPAD, H), _f32),
    )(bexp, bval, xs, W1, b1r, W2, b2r, ws.reshape(NPAD, 1))

    combine = pl.kernel(
        _sc_combine_kernel,
        out_type=jax.ShapeDtypeStruct((T, H), _f32),
        mesh=plsc.VectorSubcoreMesh(**_SC_MESH),
        compiler_params=_SC_PARAMS,
        scratch_types=[
            pltpu.VMEM((32,), _i32),
            pltpu.VMEM((32,), _i32),
            pltpu.VMEM((32,), _i32),
            pltpu.VMEM((32,), _i32),
            pltpu.VMEM((32, H), _f32),
            pltpu.VMEM((32, H), _f32),
            pltpu.SemaphoreType.DMA,
        ],
    )
    out = combine(y, inv)
    return out


def kernel(x, Wr1, br1, Wr2, br2, W1, b1, W2, b2):
    xf = x.reshape(T, H)
    wr2t = Wr2.T
    br2c = br2.reshape(E, 1)
    br1r = br1.reshape(1, H)
    b1r = b1.reshape(E * NF, 1, FFT)
    b2r = b2.reshape(E, 1, H)
    out = _sparse_moe(xf, Wr1, br1r, wr2t, br2c, W1, b1r, W2, b2r)
    return out.reshape(B, S, H)


# speedup vs baseline: 1.1598x; 1.0044x over previous
import functools

import jax
import jax.numpy as jnp
from jax import lax
from jax.experimental import pallas as pl
from jax.experimental.pallas import tpu as pltpu
from jax.experimental.pallas import tpu_sc as plsc

B, S, H, FF, E, K = 1, 2048, 1024, 4096, 8, 2
T = B * S
TB = 256
NTB = T // TB
FFT = 512
NF = FF // FFT
EP = 128

BT = 256
NB = 32
NPAD = NB * BT
NW = 32
TT = T // NW
SLOT_T = NPAD // NW
L = 16

_f32 = jnp.float32
_i32 = jnp.int32


def _router_kernel(x_ref, wr1_ref, br1_ref, wr2t_ref, br2c_ref, lgt_ref):
    xb = x_ref[...]
    rh = jnp.maximum(
        jnp.dot(xb, wr1_ref[...], preferred_element_type=_f32)
        + br1_ref[...], 0.0)
    lgt = lax.dot_general(wr2t_ref[...], rh, (((1,), (1,)), ((), ())),
                          preferred_element_type=_f32)
    lgt_ref[...] = lgt + br2c_ref[...]


def _lane_iota():
    return lax.broadcasted_iota(_i32, (L,), 0)


def _lane_val(vec, e):
    return jnp.sum(jnp.where(_lane_iota() == e, vec, 0))


def _splat(x, dtype):
    return jnp.zeros((L,), dtype) + x


def _sc_route_kernel(lgt_hbm, x_hbm,
                     ws_hbm, inv_hbm, bexp_hbm, bval_hbm, xs_hbm,
                     lg_v, e1_v, e2_v, w1_v, w2_v, d1_v, d2_v,
                     st_v, wsl_v, bexp_v, bval_v, h1_v, h2_v, base_v,
                     gbufa_v, gbufb_v, sem, wsem):
    wid = lax.axis_index("s") * 2 + lax.axis_index("c")
    lane = _lane_iota()
    NCH = T // L

    def _zero(j, _):
        st_v[pl.ds(j * L, L)] = jnp.zeros((L,), _i32)
        wsl_v[pl.ds(j * L, L)] = jnp.zeros((L,), _f32)
        return _
    lax.fori_loop(0, NPAD // L, _zero, 0)

    pltpu.sync_copy(lgt_hbm, lg_v)

    neg = jnp.float32(-1e30)

    def _top2(c, cnt):
        ls = [lg_v[pl.ds(e * T + c * L, L)] for e in range(E)]
        m1 = ls[0]
        for e in range(1, E):
            m1 = jnp.maximum(m1, ls[e])
        i1 = jnp.full((L,), E, _i32)
        for e in reversed(range(E)):
            i1 = jnp.where(ls[e] == m1, e, i1)
        m2 = jnp.full((L,), neg, _f32)
        for e in range(E):
            m2 = jnp.maximum(m2, jnp.where(i1 == e, neg, ls[e]))
        i2 = jnp.full((L,), E, _i32)
        for e in reversed(range(E)):
            i2 = jnp.where((ls[e] == m2) & (i1 != e), e, i2)
        w1 = 1.0 / (1.0 + jnp.exp(m2 - m1))
        w2 = 1.0 - w1
        e1_v[pl.ds(c * L, L)] = i1
        e2_v[pl.ds(c * L, L)] = i2
        w1_v[pl.ds(c * L, L)] = w1
        w2_v[pl.ds(c * L, L)] = w2
        h1 = jnp.zeros((L,), _i32)
        h2 = jnp.zeros((L,), _i32)
        for e in range(E):
            n1 = jnp.sum(jnp.where(i1 == e, 1, 0))
            n2 = jnp.sum(jnp.where(i2 == e, 1, 0))
            h1 = h1 + jnp.where(lane == e, _splat(n1, _i32), 0)
            h2 = h2 + jnp.where(lane == e, _splat(n2, _i32), 0)
        h1_v[pl.ds(c * L, L)] = h1
        h2_v[pl.ds(c * L, L)] = h2
        return cnt + h1 + h2

    total = lax.fori_loop(0, NCH, _top2, jnp.zeros((L,), _i32))

    padded = (total + (BT - 1)) & ~(BT - 1)
    incl = plsc.cumsum(padded)
    excl = incl - padded
    tp = jnp.max(incl)

    tpv = _splat(tp, _i32)
    for j in range(NB // L):
        bstart = (lane + j * L) * BT
        eb = jnp.zeros((L,), _i32)
        for e in range(E):
            incl_e = _splat(_lane_val(incl, e), _i32)
            eb = eb + jnp.where(bstart >= incl_e, 1, 0)
        eb = jnp.minimum(eb, E - 1)
        bexp_v[pl.ds(j * L, L)] = eb
        bval_v[pl.ds(j * L, L)] = jnp.where(bstart < tpv, 1, 0)

    @pl.when(wid == 0)
    def _():
        pltpu.sync_copy(bexp_v, bexp_hbm)
        pltpu.sync_copy(bval_v, bval_hbm)

    def _dst(c, base):
        tok = c * L + lane
        for (ev_ref, wv_ref, dv_ref, hist_ref) in (
                (e1_v, w1_v, d1_v, h1_v), (e2_v, w2_v, d2_v, h2_v)):
            ev = ev_ref[pl.ds(c * L, L)]
            wv = wv_ref[pl.ds(c * L, L)]
            base_v[...] = base
            bsel = plsc.load_gather(base_v, [ev])
            pos = jnp.zeros((L,), _i32)
            for e in range(E):
                m = ev == e
                mi = jnp.where(m, 1, 0)
                pos = pos + jnp.where(m, plsc.cumsum(mi) - 1, 0)
            dst = bsel + pos
            dv_ref[pl.ds(c * L, L)] = dst
            plsc.store_scatter(st_v, [dst], tok)
            plsc.store_scatter(wsl_v, [dst], wv)
            base = base + hist_ref[pl.ds(c * L, L)]
        return base

    lax.fori_loop(0, NCH, _dst, excl)

    pltpu.sync_copy(d1_v.at[pl.ds(wid * TT, TT)],
                    inv_hbm.at[0, pl.ds(wid * TT, TT)])
    pltpu.sync_copy(d2_v.at[pl.ds(wid * TT, TT)],
                    inv_hbm.at[1, pl.ds(wid * TT, TT)])
    pltpu.sync_copy(wsl_v.at[pl.ds(wid * SLOT_T, SLOT_T)],
                    ws_hbm.at[pl.ds(wid * SLOT_T, SLOT_T)])

    for g in range(SLOT_T // 64):
        start = wid * SLOT_T + g * 64

        @pl.when(start < tp)
        def _():
            cpa = pltpu.async_copy(x_hbm.at[st_v.at[pl.ds(start, 32)]],
                                   gbufa_v, sem)
            cpb = pltpu.async_copy(x_hbm.at[st_v.at[pl.ds(start + 32, 32)]],
                                   gbufb_v, sem)
            cpa.wait()
            wa = pltpu.async_copy(gbufa_v, xs_hbm.at[pl.ds(start, 32)], wsem)
            cpb.wait()
            wb = pltpu.async_copy(gbufb_v, xs_hbm.at[pl.ds(start + 32, 32)],
                                  wsem)
            wa.wait()
            wb.wait()


def _group_ffn_kernel(bexp_s, bval_s, xs_ref, w1_ref, b1_ref, w2_ref,
                      b2_ref, ws_ref, y_ref):
    b = pl.program_id(0)
    f = pl.program_id(1)

    @pl.when(bval_s[b] == 1)
    def _():
        h1 = (jnp.dot(xs_ref[...], w1_ref[0], preferred_element_type=_f32)
              + b1_ref[0])
        h1 = 0.5 * h1 * (1.0 + lax.erf(h1 * 0.7071067811865476))
        partial = jnp.dot(h1, w2_ref[0], preferred_element_type=_f32)

        @pl.when(f == 0)
        def _():
            y_ref[...] = partial

        @pl.when(f != 0)
        def _():
            y_ref[...] += partial

        @pl.when(f == NF - 1)
        def _():
            y_ref[...] = (y_ref[...] + b2_ref[0]) * ws_ref[...]


def _sc_combine_kernel(y_hbm, inv_hbm, out_hbm,
                       i0a_v, i0b_v, i1a_v, i1b_v, g0_v, g1_v, sem):
    wid = lax.axis_index("s") * 2 + lax.axis_index("c")
    t0 = wid * TT
    pltpu.sync_copy(inv_hbm.at[0, pl.ds(t0, 32)], i0a_v)
    pltpu.sync_copy(inv_hbm.at[0, pl.ds(t0 + 32, 32)], i0b_v)
    pltpu.sync_copy(inv_hbm.at[1, pl.ds(t0, 32)], i1a_v)
    pltpu.sync_copy(inv_hbm.at[1, pl.ds(t0 + 32, 32)], i1b_v)
    for g, (ia, ib) in enumerate(((i0a_v, i1a_v), (i0b_v, i1b_v))):
        pltpu.async_copy(y_hbm.at[ia], g0_v, sem).wait()
        pltpu.async_copy(y_hbm.at[ib], g1_v, sem).wait()
        for r in range(32):
            def _body(cc, _):
                for u in range(4):
                    sl = pl.ds(cc * 4 * L + u * L, L)
                    g0_v[r, sl] += g1_v[r, sl]
                return _
            lax.fori_loop(0, H // (4 * L), _body, 0)
        pltpu.sync_copy(g0_v, out_hbm.at[pl.ds(t0 + g * 32, 32)])


_SC_MESH = dict(core_axis_name="c", subcore_axis_name="s",
                num_cores=2, num_subcores=16)
_SC_PARAMS = pltpu.CompilerParams(needs_layout_passes=False)


def _sparse_moe(xf, Wr1, br1r, wr2t, br2c, W1, b1r, W2, b2r):
    lgt = pl.pallas_call(
        _router_kernel,
        grid=(NTB,),
        in_specs=[
            pl.BlockSpec((TB, H), lambda i: (i, 0)),
            pl.BlockSpec((H, H), lambda i: (0, 0)),
            pl.BlockSpec((1, H), lambda i: (0, 0)),
            pl.BlockSpec((E, H), lambda i: (0, 0)),
            pl.BlockSpec((E, 1), lambda i: (0, 0)),
        ],
        out_specs=pl.BlockSpec((E, TB), lambda i: (0, i)),
        out_shape=jax.ShapeDtypeStruct((E, T), _f32),
    )(xf, Wr1, br1r, wr2t, br2c)

    route = pl.kernel(
        _sc_route_kernel,
        out_type=(
            jax.ShapeDtypeStruct((NPAD,), _f32),
            jax.ShapeDtypeStruct((2, T), _i32),
            jax.ShapeDtypeStruct((NB,), _i32),
            jax.ShapeDtypeStruct((NB,), _i32),
            jax.ShapeDtypeStruct((NPAD, H), _f32),
        ),
        mesh=plsc.VectorSubcoreMesh(**_SC_MESH),
        compiler_params=_SC_PARAMS,
        scratch_types=[
            pltpu.VMEM((E * T,), _f32),
            pltpu.VMEM((T,), _i32),
            pltpu.VMEM((T,), _i32),
            pltpu.VMEM((T,), _f32),
            pltpu.VMEM((T,), _f32),
            pltpu.VMEM((T,), _i32),
            pltpu.VMEM((T,), _i32),
            pltpu.VMEM((NPAD,), _i32),
            pltpu.VMEM((NPAD,), _f32),
            pltpu.VMEM((NB,), _i32),
            pltpu.VMEM((NB,), _i32),
            pltpu.VMEM((T,), _i32),
            pltpu.VMEM((T,), _i32),
            pltpu.VMEM((L,), _i32),
            pltpu.VMEM((32, H), _f32),
            pltpu.VMEM((32, H), _f32),
            pltpu.SemaphoreType.DMA,
            pltpu.SemaphoreType.DMA,
        ],
    )
    ws, inv, bexp, bval, xs = route(lgt.reshape(E * T), xf)

    grid_spec = pltpu.PrefetchScalarGridSpec(
        num_scalar_prefetch=2,
        grid=(NB, NF),
        in_specs=[
            pl.BlockSpec((BT, H), lambda b, f, be, bv: (b * bv[b], 0)),
            pl.BlockSpec((1, H, FFT), lambda b, f, be, bv: (be[b], 0, f * bv[b])),
            pl.BlockSpec((1, 1, FFT),
                         lambda b, f, be, bv: (be[b] * NF + f * bv[b], 0, 0)),
            pl.BlockSpec((1, FFT, H), lambda b, f, be, bv: (be[b], f * bv[b], 0)),
            pl.BlockSpec((1, 1, H), lambda b, f, be, bv: (be[b], 0, 0)),
            pl.BlockSpec((BT, 1), lambda b, f, be, bv: (b * bv[b], 0)),
        ],
        out_specs=pl.BlockSpec((BT, H), lambda b, f, be, bv: (b, 0)),
    )
    y = pl.pallas_call(
        _group_ffn_kernel,
        grid_spec=grid_spec,
        out_shape=jax.ShapeDtypeStruct((NPAD, H), _f32),
    )(bexp, bval, xs, W1, b1r, W2, b2r, ws.reshape(NPAD, 1))

    combine = pl.kernel(
        _sc_combine_kernel,
        out_type=jax.ShapeDtypeStruct((T, H), _f32),
        mesh=plsc.VectorSubcoreMesh(**_SC_MESH),
        compiler_params=_SC_PARAMS,
        scratch_types=[
            pltpu.VMEM((32,), _i32),
            pltpu.VMEM((32,), _i32),
            pltpu.VMEM((32,), _i32),
            pltpu.VMEM((32,), _i32),
            pltpu.VMEM((32, H), _f32),
            pltpu.VMEM((32, H), _f32),
            pltpu.SemaphoreType.DMA,
        ],
    )
    out = combine(y, inv)
    return out


def kernel(x, Wr1, br1, Wr2, br2, W1, b1, W2, b2):
    xf = x.reshape(T, H)
    wr2t = Wr2.T
    br2c = br2.reshape(E, 1)
    br1r = br1.reshape(1, H)
    b1r = b1.reshape(E * NF, 1, FFT)
    b2r = b2.reshape(E, 1, H)
    out = _sparse_moe(xf, Wr1, br1r, wr2t, br2c, W1, b1r, W2, b2r)
    return out.reshape(B, S, H)
